# baseline (speedup 1.0000x reference)
import functools

import jax
import jax.numpy as jnp
from jax.experimental import pallas as pl
from jax.experimental.pallas import tpu as pltpu
from jax.experimental.pallas import tpu_sc as plsc

_E = 8
_K = 2
_LANES = 16
_NW = 32


def _gelu_exact(v):
    return 0.5 * v * (1.0 + jax.lax.erf(v * 0.7071067811865476))


def _logits_body(rh_ref, wr_ref, o_ref):
    o_ref[...] = jnp.dot(rh_ref[...], wr_ref[...],
                         preferred_element_type=jnp.float32)


def _make_router_sc(N):
    PT = N // _NW
    mesh = plsc.VectorSubcoreMesh(core_axis_name="c", subcore_axis_name="s")

    @functools.partial(
        pl.kernel, mesh=mesh,
        out_type=jax.ShapeDtypeStruct((_E, N), jnp.float32),
        scratch_types=[
            pltpu.VMEM((_E, PT), jnp.float32),
            pltpu.VMEM((_E, PT), jnp.float32),
        ],
    )
    def rkern(lg_hbm, w_hbm, lbuf, wbuf):
        cid = jax.lax.axis_index("c")
        sid = jax.lax.axis_index("s")
        wid = sid * 2 + cid
        base = wid * PT
        for e in range(_E):
            pltpu.sync_copy(lg_hbm.at[e, pl.ds(base, PT)], lbuf.at[e])

        def chunk(ci, carry):
            sl = pl.ds(ci * _LANES, _LANES)
            cols = [lbuf[e, sl] for e in range(_E)]
            m1 = cols[0]
            for e in range(1, _E):
                m1 = jnp.maximum(m1, cols[e])
            e1 = jnp.full((_LANES,), _E, jnp.int32)
            for e in range(_E - 1, -1, -1):
                e1 = jnp.where(cols[e] == m1, e, e1)
            m2 = jnp.full((_LANES,), -1e30, jnp.float32)
            for e in range(_E):
                m2 = jnp.maximum(m2, jnp.where(e1 == e, -1e30, cols[e]))
            e2 = jnp.full((_LANES,), _E, jnp.int32)
            for e in range(_E - 1, -1, -1):
                e2 = jnp.where((e1 != e) & (cols[e] == m2), e, e2)
            w1 = 1.0 / (1.0 + jnp.exp(m2 - m1))
            w2 = 1.0 - w1
            for e in range(_E):
                wbuf[e, sl] = jnp.where(e1 == e, w1,
                                        jnp.where(e2 == e, w2, 0.0))
            return carry

        jax.lax.fori_loop(0, PT // _LANES, chunk, 0)
        for e in range(_E):
            pltpu.sync_copy(wbuf.at[e], w_hbm.at[e, pl.ds(base, PT)])

    return rkern


def _moe_body(x_ref, w_ref, wd_ref, wu_ref, out_ref, o_ref):
    xb = x_ref[...].astype(jnp.bfloat16)
    o_ref[...] = out_ref[...]
    for e in range(_E):
        onehot = (jax.lax.broadcasted_iota(jnp.int32, (_E, 1), 0) == e
                  ).astype(jnp.float32)
        we = jnp.dot(w_ref[...], onehot, preferred_element_type=jnp.float32)
        h = jnp.dot(xb, wd_ref[e], preferred_element_type=jnp.float32)
        g = (_gelu_exact(h) * we).astype(jnp.bfloat16)
        o_ref[...] = o_ref[...] + jnp.dot(g, wu_ref[e],
                                          preferred_element_type=jnp.float32)


def kernel(input_hidden_states, output_hidden_states, router_hidden_states,
           Wr, Wd, Wu):
    orig_shape = output_hidden_states.shape
    D = orig_shape[-1]
    x = input_hidden_states.reshape(-1, D)
    out = output_hidden_states.reshape(-1, D)
    rh = router_hidden_states.reshape(-1, D)
    N = x.shape[0]
    A = Wd.shape[2]
    EA = _E * A

    Wd16 = Wd.astype(jnp.bfloat16)
    Wu16 = Wu.astype(jnp.bfloat16)

    T0 = 1024
    logits = pl.pallas_call(
        _logits_body,
        grid=(N // T0,),
        in_specs=[
            pl.BlockSpec((T0, D), lambda i: (i, 0)),
            pl.BlockSpec((D, _E), lambda i: (0, 0)),
        ],
        out_specs=pl.BlockSpec((T0, _E), lambda i: (i, 0)),
        out_shape=jax.ShapeDtypeStruct((N, _E), jnp.float32),
    )(rh, Wr)

    W = _make_router_sc(N)(logits.T).T

    T = 256
    res = pl.pallas_call(
        _moe_body,
        grid=(N // T,),
        in_specs=[
            pl.BlockSpec((T, D), lambda i: (i, 0)),
            pl.BlockSpec((T, _E), lambda i: (i, 0)),
            pl.BlockSpec((_E, D, A), lambda i: (0, 0, 0)),
            pl.BlockSpec((_E, A, D), lambda i: (0, 0, 0)),
            pl.BlockSpec((T, D), lambda i: (i, 0)),
        ],
        out_specs=pl.BlockSpec((T, D), lambda i: (i, 0)),
        out_shape=jax.ShapeDtypeStruct((N, D), jnp.float32),
    )(x, W, Wd16, Wu16, out)
    return res.reshape(orig_shape)


# speedup vs baseline: 1.0457x; 1.0457x over previous
import functools

import jax
import jax.numpy as jnp
from jax.experimental import pallas as pl
from jax.experimental.pallas import tpu as pltpu
from jax.experimental.pallas import tpu_sc as plsc

_E = 8
_K = 2
_LANES = 16
_NW = 32


def _gelu_exact(v):
    return 0.5 * v * (1.0 + jax.lax.erf(v * 0.7071067811865476))


def _logits_body(rh_ref, wr_ref, o_ref):
    o_ref[...] = jnp.dot(rh_ref[...], wr_ref[...],
                         preferred_element_type=jnp.float32)


def _make_router_sc(N):
    PT = N // _NW
    mesh = plsc.VectorSubcoreMesh(core_axis_name="c", subcore_axis_name="s")

    @functools.partial(
        pl.kernel, mesh=mesh,
        out_type=jax.ShapeDtypeStruct((_E, N), jnp.float32),
        scratch_types=[
            pltpu.VMEM((_E, PT), jnp.float32),
            pltpu.VMEM((_E, PT), jnp.float32),
        ],
    )
    def rkern(lg_hbm, w_hbm, lbuf, wbuf):
        cid = jax.lax.axis_index("c")
        sid = jax.lax.axis_index("s")
        wid = sid * 2 + cid
        base = wid * PT
        for e in range(_E):
            pltpu.sync_copy(lg_hbm.at[e, pl.ds(base, PT)], lbuf.at[e])

        def chunk(ci, carry):
            sl = pl.ds(ci * _LANES, _LANES)
            cols = [lbuf[e, sl] for e in range(_E)]
            m1 = cols[0]
            for e in range(1, _E):
                m1 = jnp.maximum(m1, cols[e])
            e1 = jnp.full((_LANES,), _E, jnp.int32)
            for e in range(_E - 1, -1, -1):
                e1 = jnp.where(cols[e] == m1, e, e1)
            m2 = jnp.full((_LANES,), -1e30, jnp.float32)
            for e in range(_E):
                m2 = jnp.maximum(m2, jnp.where(e1 == e, -1e30, cols[e]))
            e2 = jnp.full((_LANES,), _E, jnp.int32)
            for e in range(_E - 1, -1, -1):
                e2 = jnp.where((e1 != e) & (cols[e] == m2), e, e2)
            w1 = 1.0 / (1.0 + jnp.exp(m2 - m1))
            w2 = 1.0 - w1
            for e in range(_E):
                wbuf[e, sl] = jnp.where(e1 == e, w1,
                                        jnp.where(e2 == e, w2, 0.0))
            return carry

        jax.lax.fori_loop(0, PT // _LANES, chunk, 0)
        for e in range(_E):
            pltpu.sync_copy(wbuf.at[e], w_hbm.at[e, pl.ds(base, PT)])

    return rkern


def _down_body(x_ref, w_ref, wd_ref, z_ref):
    e = pl.program_id(1)
    onehot = (jax.lax.broadcasted_iota(jnp.int32, (_E, 1), 0) == e
              ).astype(jnp.float32)
    we = jnp.dot(w_ref[...], onehot, preferred_element_type=jnp.float32)
    h = jnp.dot(x_ref[...].astype(jnp.bfloat16),
                wd_ref[0].astype(jnp.bfloat16),
                preferred_element_type=jnp.float32)
    z_ref[...] = (_gelu_exact(h) * we).astype(jnp.bfloat16)


def _up_body(z_ref, wu_ref, out_ref, o_ref):
    o_ref[...] = jnp.dot(z_ref[...], wu_ref[...],
                         preferred_element_type=jnp.float32) + out_ref[...]


def kernel(input_hidden_states, output_hidden_states, router_hidden_states,
           Wr, Wd, Wu):
    orig_shape = output_hidden_states.shape
    D = orig_shape[-1]
    x = input_hidden_states.reshape(-1, D)
    out = output_hidden_states.reshape(-1, D)
    rh = router_hidden_states.reshape(-1, D)
    N = x.shape[0]
    A = Wd.shape[2]
    EA = _E * A

    Wu16 = Wu.reshape(EA, D).astype(jnp.bfloat16)

    T0 = 1024
    logits = pl.pallas_call(
        _logits_body,
        grid=(N // T0,),
        in_specs=[
            pl.BlockSpec((T0, D), lambda i: (i, 0)),
            pl.BlockSpec((D, _E), lambda i: (0, 0)),
        ],
        out_specs=pl.BlockSpec((T0, _E), lambda i: (i, 0)),
        out_shape=jax.ShapeDtypeStruct((N, _E), jnp.float32),
    )(rh, Wr)

    W = _make_router_sc(N)(logits.T).T

    T2 = 2048
    z = pl.pallas_call(
        _down_body,
        grid=(N // T2, _E),
        in_specs=[
            pl.BlockSpec((T2, D), lambda i, e: (i, 0)),
            pl.BlockSpec((T2, _E), lambda i, e: (i, 0)),
            pl.BlockSpec((1, D, A), lambda i, e: (e, 0, 0)),
        ],
        out_specs=pl.BlockSpec((T2, A), lambda i, e: (i, e)),
        out_shape=jax.ShapeDtypeStruct((N, EA), jnp.bfloat16),
    )(x, W, Wd)

    T3 = 512
    res = pl.pallas_call(
        _up_body,
        grid=(N // T3,),
        in_specs=[
            pl.BlockSpec((T3, EA), lambda i: (i, 0)),
            pl.BlockSpec((EA, D), lambda i: (0, 0)),
            pl.BlockSpec((T3, D), lambda i: (i, 0)),
        ],
        out_specs=pl.BlockSpec((T3, D), lambda i: (i, 0)),
        out_shape=jax.ShapeDtypeStruct((N, D), jnp.float32),
    )(z, Wu16, out)
    return res.reshape(orig_shape)
